# baseline (speedup 1.0000x reference)
import functools

import jax
import jax.numpy as jnp
from jax import lax
from jax.experimental import pallas as pl
from jax.experimental.pallas import tpu as pltpu
from jax.experimental.pallas import tpu_sc as plsc

NUM_MONTHS = 12
EMBED = 128
BATCH = 16384

_NC = 2
_NS = 16
_NW = _NC * _NS
_BPW = BATCH // _NW
_ROWS_PER_STEP = 8


def _make_kernel():
  mesh = plsc.VectorSubcoreMesh(core_axis_name="c", subcore_axis_name="s")

  @functools.partial(
      pl.kernel,
      mesh=mesh,
      out_type=jax.ShapeDtypeStruct((BATCH * EMBED,), jnp.float32),
      scratch_types=[
          pltpu.VMEM((NUM_MONTHS * EMBED,), jnp.float32),
          pltpu.VMEM((_BPW * 2,), jnp.int32),
          pltpu.VMEM((_BPW * EMBED,), jnp.float32),
          pltpu.SemaphoreType.DMA,
          pltpu.SemaphoreType.DMA,
      ],
  )
  def k(ti_hbm, table_hbm, out_hbm, table_v, ti_v, drain_v, in_sem, out_sem):
    wid = lax.axis_index("s") * _NC + lax.axis_index("c")
    base = wid * _BPW

    load_table = pltpu.async_copy(table_hbm, table_v, in_sem)
    pltpu.sync_copy(ti_hbm.at[pl.ds(base * 2, _BPW * 2)], ti_v)
    load_table.wait()

    @plsc.parallel_loop(0, _BPW, step=_ROWS_PER_STEP, unroll=2)
    def _(r0):
      pairs = ti_v[pl.ds(r0 * 2, 2 * _ROWS_PER_STEP)]
      for r in range(_ROWS_PER_STEP):
        off = pairs[2 * r] * EMBED
        dst = (base + r0 + r) * EMBED
        pltpu.async_copy(
            table_v.at[pl.ds(off, EMBED)],
            out_hbm.at[pl.ds(dst, EMBED)],
            out_sem,
        )

    pltpu.make_async_copy(
        out_hbm.at[pl.ds(base * EMBED, _BPW * EMBED)], drain_v, out_sem
    ).wait()

  return k


_sc_lookup = jax.jit(_make_kernel())


def kernel(time_input, month_table):
  out = _sc_lookup(
      time_input.astype(jnp.int32).reshape(-1), month_table.reshape(-1)
  )
  return out.reshape(BATCH, EMBED)


# speedup vs baseline: 1.0158x; 1.0158x over previous
import functools

import jax
import jax.numpy as jnp
from jax import lax
from jax.experimental import pallas as pl
from jax.experimental.pallas import tpu as pltpu
from jax.experimental.pallas import tpu_sc as plsc

NUM_MONTHS = 12
EMBED = 128
BATCH = 16384

_NC = 2
_NS = 16
_NW = _NC * _NS
_BPW = BATCH // _NW
_ROWS_PER_STEP = 8


def _make_kernel():
  mesh = plsc.VectorSubcoreMesh(core_axis_name="c", subcore_axis_name="s")

  @functools.partial(
      pl.kernel,
      mesh=mesh,
      out_type=jax.ShapeDtypeStruct((BATCH * EMBED,), jnp.float32),
      scratch_types=[
          pltpu.VMEM((NUM_MONTHS * EMBED,), jnp.float32),
          pltpu.VMEM((_BPW * 2,), jnp.int32),
          pltpu.VMEM((_BPW * EMBED,), jnp.float32),
          pltpu.SemaphoreType.DMA,
          pltpu.SemaphoreType.DMA,
      ],
  )
  def k(ti_hbm, table_hbm, out_hbm, table_v, ti_v, drain_v, in_sem, out_sem):
    wid = lax.axis_index("s") * _NC + lax.axis_index("c")
    base = wid * _BPW

    load_table = pltpu.async_copy(table_hbm, table_v, in_sem)
    pltpu.sync_copy(ti_hbm.at[pl.ds(base * 2, _BPW * 2)], ti_v)
    load_table.wait()

    def body(step, carry):
      r0 = step * _ROWS_PER_STEP
      pairs = ti_v[pl.ds(r0 * 2, 2 * _ROWS_PER_STEP)]
      for r in range(_ROWS_PER_STEP):
        off = pairs[2 * r] * EMBED
        dst = (base + r0 + r) * EMBED
        pltpu.async_copy(
            table_v.at[pl.ds(off, EMBED)],
            out_hbm.at[pl.ds(dst, EMBED)],
            out_sem,
        )
      return carry

    lax.fori_loop(0, _BPW // _ROWS_PER_STEP, body, 0, unroll=False)

    pltpu.make_async_copy(
        out_hbm.at[pl.ds(base * EMBED, _BPW * EMBED)], drain_v, out_sem
    ).wait()

  return k


_sc_lookup = jax.jit(_make_kernel())


def kernel(time_input, month_table):
  out = _sc_lookup(
      time_input.astype(jnp.int32).reshape(-1), month_table.reshape(-1)
  )
  return out.reshape(BATCH, EMBED)


# speedup vs baseline: 1.0553x; 1.0388x over previous
import functools

import jax
import jax.numpy as jnp
from jax import lax
from jax.experimental import pallas as pl
from jax.experimental.pallas import tpu as pltpu
from jax.experimental.pallas import tpu_sc as plsc

NUM_MONTHS = 12
EMBED = 128
BATCH = 16384

_NC = 1
_NS = 16
_NW = _NC * _NS
_BPW = BATCH // _NW
_ROWS_PER_STEP = 8


def _make_kernel():
  mesh = plsc.VectorSubcoreMesh(core_axis_name="c", subcore_axis_name="s", num_cores=1)

  @functools.partial(
      pl.kernel,
      mesh=mesh,
      out_type=jax.ShapeDtypeStruct((BATCH * EMBED,), jnp.float32),
      scratch_types=[
          pltpu.VMEM((NUM_MONTHS * EMBED,), jnp.float32),
          pltpu.VMEM((_BPW * 2,), jnp.int32),
          pltpu.VMEM((_BPW * EMBED // 2,), jnp.float32),
          pltpu.SemaphoreType.DMA,
          pltpu.SemaphoreType.DMA,
      ],
  )
  def k(ti_hbm, table_hbm, out_hbm, table_v, ti_v, drain_v, in_sem, out_sem):
    wid = lax.axis_index("s") * _NC + lax.axis_index("c")
    base = wid * _BPW

    load_table = pltpu.async_copy(table_hbm, table_v, in_sem)
    pltpu.sync_copy(ti_hbm.at[pl.ds(base * 2, _BPW * 2)], ti_v)
    load_table.wait()

    def body(step, carry):
      r0 = step * _ROWS_PER_STEP
      pairs = ti_v[pl.ds(r0 * 2, 2 * _ROWS_PER_STEP)]
      for r in range(_ROWS_PER_STEP):
        off = pairs[2 * r] * EMBED
        dst = (base + r0 + r) * EMBED
        pltpu.async_copy(
            table_v.at[pl.ds(off, EMBED)],
            out_hbm.at[pl.ds(dst, EMBED)],
            out_sem,
        )
      return carry

    lax.fori_loop(0, _BPW // _ROWS_PER_STEP, body, 0, unroll=False)

    for _h in range(2):
      pltpu.make_async_copy(
          out_hbm.at[pl.ds(base * EMBED, _BPW * EMBED // 2)], drain_v, out_sem
      ).wait()

  return k


_sc_lookup = jax.jit(_make_kernel())


def kernel(time_input, month_table):
  out = _sc_lookup(
      time_input.astype(jnp.int32).reshape(-1), month_table.reshape(-1)
  )
  return out.reshape(BATCH, EMBED)
